# baseline (device time: 1451507 ns/iter reference)
import jax
import jax.numpy as jnp
from jax import lax
from jax.experimental import pallas as pl
from jax.experimental.pallas import tpu as pltpu

N_DEV = 16


def kernel(x, w_mat):
    m_glob, k_loc = x.shape
    _, n = w_mat.shape
    m_chunk = m_glob // N_DEV

    def body(x_ref, w_ref, out_ref, send_ref, recv_ref,
             send_sem, recv_sem, credit_sem):
        p = lax.axis_index("i")
        left = lax.rem(p + N_DEV - 1, N_DEV)
        right = lax.rem(p + 1, N_DEV)

        barrier_sem = pltpu.get_barrier_semaphore()
        for nbr in (left, right):
            pl.semaphore_signal(
                barrier_sem, inc=1,
                device_id=(nbr,), device_id_type=pl.DeviceIdType.MESH,
            )
        pl.semaphore_wait(barrier_sem, 2)

        def compute_chunk(idx):
            xs = x_ref[pl.ds(idx * m_chunk, m_chunk), :]
            return jnp.dot(xs, w_ref[:, :], preferred_element_type=jnp.float32)

        send_ref[:, :] = compute_chunk(lax.rem(p + N_DEV - 1, N_DEV))

        for s in range(N_DEV - 1):
            if s > 0:
                pl.semaphore_wait(credit_sem, 1)
            rdma = pltpu.make_async_remote_copy(
                src_ref=send_ref,
                dst_ref=recv_ref,
                send_sem=send_sem,
                recv_sem=recv_sem,
                device_id=(right,),
                device_id_type=pl.DeviceIdType.MESH,
            )
            rdma.start()
            idx = lax.rem(p + N_DEV - 2 - s, N_DEV)
            part = compute_chunk(idx)
            rdma.wait()
            acc = recv_ref[:, :] + part
            if s < N_DEV - 2:
                send_ref[:, :] = acc
                pl.semaphore_signal(
                    credit_sem, inc=1,
                    device_id=(left,), device_id_type=pl.DeviceIdType.MESH,
                )
            else:
                out_ref[:, :] = jnp.maximum(acc, 0.0)

    return pl.pallas_call(
        body,
        out_shape=jax.ShapeDtypeStruct((m_chunk, n), jnp.float32),
        in_specs=[
            pl.BlockSpec(memory_space=pltpu.VMEM),
            pl.BlockSpec(memory_space=pltpu.VMEM),
        ],
        out_specs=pl.BlockSpec(memory_space=pltpu.VMEM),
        scratch_shapes=[
            pltpu.VMEM((m_chunk, n), jnp.float32),
            pltpu.VMEM((m_chunk, n), jnp.float32),
            pltpu.SemaphoreType.DMA,
            pltpu.SemaphoreType.DMA,
            pltpu.SemaphoreType.REGULAR,
        ],
        compiler_params=pltpu.CompilerParams(collective_id=0),
    )(x, w_mat)


# device time: 776263 ns/iter; 1.8699x vs baseline; 1.8699x over previous
import jax
import jax.numpy as jnp
from jax import lax
from jax.experimental import pallas as pl
from jax.experimental.pallas import tpu as pltpu

N_DEV = 16


def kernel(x, w_mat):
    m_glob, k_loc = x.shape
    _, n = w_mat.shape
    m_chunk = m_glob // N_DEV
    half = n // 2

    def body(x_ref, w_ref, out_ref,
             sendf_ref, recvf_ref, sendb_ref, recvb_ref,
             sendf_sem, recvf_sem, sendb_sem, recvb_sem,
             creditf_sem, creditb_sem):
        p = lax.axis_index("i")
        left = lax.rem(p + N_DEV - 1, N_DEV)
        right = lax.rem(p + 1, N_DEV)

        barrier_sem = pltpu.get_barrier_semaphore()
        for nbr in (left, right):
            pl.semaphore_signal(
                barrier_sem, inc=1,
                device_id=(nbr,), device_id_type=pl.DeviceIdType.MESH,
            )
        pl.semaphore_wait(barrier_sem, 2)

        def part_f(idx):
            xs = x_ref[pl.ds(idx * m_chunk, m_chunk), :]
            return jnp.dot(xs, w_ref[:, :half], preferred_element_type=jnp.float32)

        def part_b(idx):
            xs = x_ref[pl.ds(idx * m_chunk, m_chunk), :]
            return jnp.dot(xs, w_ref[:, half:], preferred_element_type=jnp.float32)

        sendf_ref[:, :] = part_f(lax.rem(p + N_DEV - 1, N_DEV))
        sendb_ref[:, :] = part_b(lax.rem(p + 1, N_DEV))

        for s in range(N_DEV - 1):
            if s > 0:
                pl.semaphore_wait(creditf_sem, 1)
                pl.semaphore_wait(creditb_sem, 1)
            rf = pltpu.make_async_remote_copy(
                src_ref=sendf_ref, dst_ref=recvf_ref,
                send_sem=sendf_sem, recv_sem=recvf_sem,
                device_id=(right,), device_id_type=pl.DeviceIdType.MESH,
            )
            rb = pltpu.make_async_remote_copy(
                src_ref=sendb_ref, dst_ref=recvb_ref,
                send_sem=sendb_sem, recv_sem=recvb_sem,
                device_id=(left,), device_id_type=pl.DeviceIdType.MESH,
            )
            rf.start()
            rb.start()
            pf = part_f(lax.rem(p + N_DEV - 2 - s, N_DEV))
            pb = part_b(lax.rem(p + s + 2, N_DEV))
            rf.wait()
            rb.wait()
            if s < N_DEV - 2:
                sendf_ref[:, :] = recvf_ref[:, :] + pf
                sendb_ref[:, :] = recvb_ref[:, :] + pb
                pl.semaphore_signal(
                    creditf_sem, inc=1,
                    device_id=(left,), device_id_type=pl.DeviceIdType.MESH,
                )
                pl.semaphore_signal(
                    creditb_sem, inc=1,
                    device_id=(right,), device_id_type=pl.DeviceIdType.MESH,
                )
            else:
                out_ref[:, :half] = jnp.maximum(recvf_ref[:, :] + pf, 0.0)
                out_ref[:, half:] = jnp.maximum(recvb_ref[:, :] + pb, 0.0)

    return pl.pallas_call(
        body,
        out_shape=jax.ShapeDtypeStruct((m_chunk, n), jnp.float32),
        in_specs=[
            pl.BlockSpec(memory_space=pltpu.VMEM),
            pl.BlockSpec(memory_space=pltpu.VMEM),
        ],
        out_specs=pl.BlockSpec(memory_space=pltpu.VMEM),
        scratch_shapes=[
            pltpu.VMEM((m_chunk, half), jnp.float32),
            pltpu.VMEM((m_chunk, half), jnp.float32),
            pltpu.VMEM((m_chunk, half), jnp.float32),
            pltpu.VMEM((m_chunk, half), jnp.float32),
            pltpu.SemaphoreType.DMA,
            pltpu.SemaphoreType.DMA,
            pltpu.SemaphoreType.DMA,
            pltpu.SemaphoreType.DMA,
            pltpu.SemaphoreType.REGULAR,
            pltpu.SemaphoreType.REGULAR,
        ],
        compiler_params=pltpu.CompilerParams(collective_id=0),
    )(x, w_mat)


# device time: 408835 ns/iter; 3.5503x vs baseline; 1.8987x over previous
import jax
import jax.numpy as jnp
from jax import lax
from jax.experimental import pallas as pl
from jax.experimental.pallas import tpu as pltpu

N_DEV = 16


def kernel(x, w_mat):
    m_glob, k_loc = x.shape
    _, n = w_mat.shape
    m_chunk = m_glob // N_DEV
    half = n // 2

    def body(x_ref, w_ref, out_ref,
             fbuf_ref, bbuf_ref,
             fsend_sems, frecv_sems, bsend_sems, brecv_sems,
             creditf_sem, creditb_sem):
        p = lax.axis_index("i")
        left = lax.rem(p + N_DEV - 1, N_DEV)
        right = lax.rem(p + 1, N_DEV)

        barrier_sem = pltpu.get_barrier_semaphore()
        for nbr in (left, right):
            pl.semaphore_signal(
                barrier_sem, inc=1,
                device_id=(nbr,), device_id_type=pl.DeviceIdType.MESH,
            )
        pl.semaphore_wait(barrier_sem, 2)

        def part_f(idx):
            xs = x_ref[pl.ds(idx * m_chunk, m_chunk), :]
            return jnp.dot(xs, w_ref[:, :half], preferred_element_type=jnp.float32)

        def part_b(idx):
            xs = x_ref[pl.ds(idx * m_chunk, m_chunk), :]
            return jnp.dot(xs, w_ref[:, half:], preferred_element_type=jnp.float32)

        fbuf_ref[0, :, :] = part_f(lax.rem(p + N_DEV - 1, N_DEV)).astype(jnp.bfloat16)
        bbuf_ref[0, :, :] = part_b(lax.rem(p + 1, N_DEV)).astype(jnp.bfloat16)

        for s in range(N_DEV - 1):
            slot = s % 2
            nslot = (s + 1) % 2
            if s > 0:
                pl.semaphore_wait(creditf_sem, 1)
                pl.semaphore_wait(creditb_sem, 1)
            rf = pltpu.make_async_remote_copy(
                src_ref=fbuf_ref.at[slot], dst_ref=fbuf_ref.at[nslot],
                send_sem=fsend_sems.at[slot], recv_sem=frecv_sems.at[nslot],
                device_id=(right,), device_id_type=pl.DeviceIdType.MESH,
            )
            rb = pltpu.make_async_remote_copy(
                src_ref=bbuf_ref.at[slot], dst_ref=bbuf_ref.at[nslot],
                send_sem=bsend_sems.at[slot], recv_sem=brecv_sems.at[nslot],
                device_id=(left,), device_id_type=pl.DeviceIdType.MESH,
            )
            rf.start()
            rb.start()
            pf = part_f(lax.rem(p + N_DEV - 2 - s, N_DEV))
            pb = part_b(lax.rem(p + s + 2, N_DEV))
            rf.wait_send()
            rb.wait_send()
            if s < N_DEV - 2:
                pl.semaphore_signal(
                    creditf_sem, inc=1,
                    device_id=(left,), device_id_type=pl.DeviceIdType.MESH,
                )
                pl.semaphore_signal(
                    creditb_sem, inc=1,
                    device_id=(right,), device_id_type=pl.DeviceIdType.MESH,
                )
            rf.wait_recv()
            rb.wait_recv()
            if s < N_DEV - 2:
                fbuf_ref[nslot, :, :] = (
                    fbuf_ref[nslot, :, :].astype(jnp.float32) + pf
                ).astype(jnp.bfloat16)
                bbuf_ref[nslot, :, :] = (
                    bbuf_ref[nslot, :, :].astype(jnp.float32) + pb
                ).astype(jnp.bfloat16)
            else:
                out_ref[:, :half] = jnp.maximum(
                    fbuf_ref[nslot, :, :].astype(jnp.float32) + pf, 0.0
                )
                out_ref[:, half:] = jnp.maximum(
                    bbuf_ref[nslot, :, :].astype(jnp.float32) + pb, 0.0
                )

    return pl.pallas_call(
        body,
        out_shape=jax.ShapeDtypeStruct((m_chunk, n), jnp.float32),
        in_specs=[
            pl.BlockSpec(memory_space=pltpu.VMEM),
            pl.BlockSpec(memory_space=pltpu.VMEM),
        ],
        out_specs=pl.BlockSpec(memory_space=pltpu.VMEM),
        scratch_shapes=[
            pltpu.VMEM((2, m_chunk, half), jnp.bfloat16),
            pltpu.VMEM((2, m_chunk, half), jnp.bfloat16),
            pltpu.SemaphoreType.DMA((2,)),
            pltpu.SemaphoreType.DMA((2,)),
            pltpu.SemaphoreType.DMA((2,)),
            pltpu.SemaphoreType.DMA((2,)),
            pltpu.SemaphoreType.REGULAR,
            pltpu.SemaphoreType.REGULAR,
        ],
        compiler_params=pltpu.CompilerParams(collective_id=0),
    )(x, w_mat)
